# Initial kernel scaffold; baseline (speedup 1.0000x reference)
import jax
import jax.numpy as jnp
from jax.experimental import pallas as pl


def kernel(features, coords, W, gamma, beta, context, Wm, bm):
    raise NotImplementedError("write your pallas kernel here")


# speedup vs baseline: 1.2219x; 1.2219x over previous
import functools

import jax
import jax.numpy as jnp
from jax import lax
from jax.experimental import pallas as pl
from jax.experimental.pallas import tpu as pltpu
from jax.experimental.pallas import tpu_sc as plsc

_N = 200000
_CIN = 32
_COUT = 64
_CTX = 256

_R = 320
_NB = _N // _R
_SROWS = 384
_OUT_PAD = _N + _SROWS

_NC = 2
_NS = 16
_NW = _NC * _NS
_CH = 128
_NCH = 49
_BPW = _CH * _NCH
_PADN = _BPW * _NW


def _gather_sorted(features, idxp):
    mesh = plsc.VectorSubcoreMesh(core_axis_name="c", subcore_axis_name="s")

    @functools.partial(
        pl.kernel,
        mesh=mesh,
        compiler_params=pltpu.CompilerParams(use_tc_tiling_on_sc=False),
        out_type=jax.ShapeDtypeStruct((_PADN, _CIN), jnp.float32),
        scratch_types=[
            pltpu.VMEM((_CH,), jnp.int32),
            pltpu.VMEM((_CH, _CIN), jnp.float32),
            pltpu.SemaphoreType.DMA,
        ],
    )
    def gk(tab_hbm, idx_hbm, out_hbm, idx_v, rows_v, sem):
        wid = lax.axis_index("s") * _NC + lax.axis_index("c")
        base = wid * _BPW

        def body(c, carry):
            off = base + c * _CH
            pltpu.sync_copy(idx_hbm.at[pl.ds(off, _CH)], idx_v)
            pltpu.async_copy(tab_hbm.at[idx_v], rows_v, sem).wait()
            pltpu.sync_copy(rows_v, out_hbm.at[pl.ds(off, _CH)])
            return carry

        lax.fori_loop(0, _NCH, body, 0)

    return gk(features, idxp)


def _conv_body(fs_ref, kv_ref, ks_ref, kc_ref, w_ref,
               out_ref, sum_ref, sq_ref, m_ref,
               s_ref, carry_ref, sm, sem):
    b = pl.program_id(0)
    nb = pl.num_programs(0)

    @pl.when(b == 0)
    def _init():
        sm[0] = ks_ref[0, 0, 0]
        sm[1] = 0
        carry_ref[...] = jnp.zeros_like(carry_ref)
        sum_ref[...] = jnp.zeros_like(sum_ref)
        sq_ref[...] = jnp.zeros_like(sq_ref)

    keys = kv_ref[0]
    prev = sm[0]
    lane = lax.broadcasted_iota(jnp.int32, (1, _R), 1)
    shifted = jnp.where(lane == 0, prev, pltpu.roll(keys, 1, axis=1))
    f = (keys != shifted).astype(jnp.float32)
    triu = (lax.broadcasted_iota(jnp.int32, (_R, _R), 0)
            <= lax.broadcasted_iota(jnp.int32, (_R, _R), 1)).astype(jnp.float32)
    local = lax.dot(f, triu, preferred_element_type=jnp.float32)
    lf = jnp.max(local)

    fblk = fs_ref[...]
    kc = kc_ref[...]
    p = jnp.zeros((_R, _COUT), jnp.float32)
    for k in range(8):
        mk = (kc == k).astype(jnp.float32)
        p = p + lax.dot(fblk * mk, w_ref[k], preferred_element_type=jnp.float32)

    lsub = lax.broadcasted_iota(jnp.int32, (_SROWS, _R), 0).astype(jnp.float32)
    a = (lsub == local).astype(jnp.float32)
    s = lax.dot(a, p, preferred_element_type=jnp.float32)
    rowi = lax.broadcasted_iota(jnp.int32, (_SROWS, 1), 0).astype(jnp.float32)
    s = s + (rowi == 0.0).astype(jnp.float32) * carry_ref[...]

    is_last = b == nb - 1
    lim = lf + jnp.where(is_last, 1.0, 0.0)
    cm = (rowi < lim).astype(jnp.float32)
    sc = s * cm
    sum_ref[...] += jnp.sum(sc, axis=0, keepdims=True)
    sq_ref[...] += jnp.sum(sc * sc, axis=0, keepdims=True)

    ml = (rowi == lf).astype(jnp.float32)
    carry_ref[...] = jnp.sum(s * ml, axis=0, keepdims=True)

    s_ref[...] = s
    base = sm[1]
    cp = pltpu.make_async_copy(s_ref, out_ref.at[pl.ds(base, _SROWS)], sem)
    cp.start()
    cp.wait()

    li = lf.astype(jnp.int32)
    sm[1] = base + li
    sm[0] = ks_ref[0, 0, _R - 1]

    @pl.when(is_last)
    def _fin():
        m_ref[0, 0] = base + li + 1


def _conv_tc(fs, keys3, kidx2, w):
    return pl.pallas_call(
        _conv_body,
        grid=(_NB,),
        in_specs=[
            pl.BlockSpec((_R, _CIN), lambda b: (b, 0)),
            pl.BlockSpec((1, 1, _R), lambda b: (b, 0, 0)),
            pl.BlockSpec((1, 1, _R), lambda b: (b, 0, 0),
                         memory_space=pltpu.SMEM),
            pl.BlockSpec((_R, 1), lambda b: (b, 0)),
            pl.BlockSpec((8, _CIN, _COUT), lambda b: (0, 0, 0)),
        ],
        out_specs=[
            pl.BlockSpec(memory_space=pl.ANY),
            pl.BlockSpec((1, _COUT), lambda b: (0, 0)),
            pl.BlockSpec((1, _COUT), lambda b: (0, 0)),
            pl.BlockSpec(memory_space=pltpu.SMEM),
        ],
        out_shape=[
            jax.ShapeDtypeStruct((_OUT_PAD, _COUT), jnp.float32),
            jax.ShapeDtypeStruct((1, _COUT), jnp.float32),
            jax.ShapeDtypeStruct((1, _COUT), jnp.float32),
            jax.ShapeDtypeStruct((1, 1), jnp.int32),
        ],
        scratch_shapes=[
            pltpu.VMEM((_SROWS, _COUT), jnp.float32),
            pltpu.VMEM((1, _COUT), jnp.float32),
            pltpu.SMEM((2,), jnp.int32),
            pltpu.SemaphoreType.DMA,
        ],
    )(fs, keys3, keys3, kidx2, w)


def _norm_body(o_ref, s1_ref, s2_ref, m_ref, g_ref, be_ref, c_ref, wm_ref,
               bm_ref, act_ref):
    b = pl.program_id(0)
    m_i = m_ref[0, 0]
    mf = m_i.astype(jnp.float32)
    mean = s1_ref[...] / mf
    var = jnp.maximum(s2_ref[...] / mf - mean * mean, 0.0)
    inv = lax.rsqrt(var + 1e-5)
    mod = lax.dot(c_ref[...], wm_ref[...],
                  preferred_element_type=jnp.float32) + bm_ref[...]
    shift = mod[:, :_COUT]
    scale = mod[:, _COUT:]
    a = (o_ref[...] - mean) * inv * g_ref[...] + be_ref[...]
    a = a * (1.0 + scale) + shift
    rows = lax.broadcasted_iota(jnp.int32, (_R, 1), 0) + b * _R
    act_ref[...] = jnp.where(rows < m_i, jnp.maximum(a, 0.0), 0.0)


def _norm_tc(outp, s1, s2, m, gamma2, beta2, ctx2, wm, bm2):
    return pl.pallas_call(
        _norm_body,
        grid=(_NB,),
        in_specs=[
            pl.BlockSpec((_R, _COUT), lambda b: (b, 0)),
            pl.BlockSpec((1, _COUT), lambda b: (0, 0)),
            pl.BlockSpec((1, _COUT), lambda b: (0, 0)),
            pl.BlockSpec(memory_space=pltpu.SMEM),
            pl.BlockSpec((1, _COUT), lambda b: (0, 0)),
            pl.BlockSpec((1, _COUT), lambda b: (0, 0)),
            pl.BlockSpec((1, _CTX), lambda b: (0, 0)),
            pl.BlockSpec((_CTX, 2 * _COUT), lambda b: (0, 0)),
            pl.BlockSpec((1, 2 * _COUT), lambda b: (0, 0)),
        ],
        out_specs=pl.BlockSpec((_R, _COUT), lambda b: (b, 0)),
        out_shape=jax.ShapeDtypeStruct((_N, _COUT), jnp.float32),
    )(outp, s1, s2, m, gamma2, beta2, ctx2, wm, bm2)


def kernel(features, coords, W, gamma, beta, context, Wm, bm):
    cb = coords[:, 0]
    cz = coords[:, 1]
    cy = coords[:, 2]
    cx = coords[:, 3]
    key = ((cb * 256 + cz // 2) * 256 + cy // 2) * 32 + cx // 2
    kidx = (cz % 2) * 4 + (cy % 2) * 2 + (cx % 2)
    iot = jnp.arange(_N, dtype=jnp.int32)
    keys_s, perm, kidx_s = lax.sort((key, iot, kidx), num_keys=1)
    padidx = jnp.arange(_PADN - _N, dtype=jnp.int32)
    permp = jnp.concatenate([perm, padidx])
    fs = _gather_sorted(features, permp)[:_N]
    keys3 = keys_s.reshape(_NB, 1, _R)
    kidx2 = kidx_s.reshape(_N, 1)
    outp, s1, s2, m = _conv_tc(fs, keys3, kidx2, W)
    act = _norm_tc(outp, s1, s2, m,
                   gamma.reshape(1, _COUT), beta.reshape(1, _COUT),
                   context.reshape(1, _CTX), Wm, bm.reshape(1, 2 * _COUT))
    return act


# speedup vs baseline: 1.5814x; 1.2942x over previous
import functools

import jax
import jax.numpy as jnp
from jax import lax
from jax.experimental import pallas as pl
from jax.experimental.pallas import tpu as pltpu
from jax.experimental.pallas import tpu_sc as plsc

_N = 200000
_CIN = 32
_COUT = 64
_CTX = 256

_R = 320
_NB = _N // _R
_SROWS = 384
_OUT_PAD = _N + _SROWS

_NC = 2
_NS = 16
_NW = _NC * _NS
_CH = 128
_NCH = 49
_BPW = _CH * _NCH
_PADN = _BPW * _NW


def _gather_sorted(features, idxp):
    mesh = plsc.VectorSubcoreMesh(core_axis_name="c", subcore_axis_name="s")

    @functools.partial(
        pl.kernel,
        mesh=mesh,
        compiler_params=pltpu.CompilerParams(use_tc_tiling_on_sc=False),
        out_type=jax.ShapeDtypeStruct((_PADN, _CIN), jnp.float32),
        scratch_types=[
            pltpu.VMEM((_CH,), jnp.int32),
            pltpu.VMEM((_CH, _CIN), jnp.float32),
            pltpu.SemaphoreType.DMA,
        ],
    )
    def gk(tab_hbm, idx_hbm, out_hbm, idx_v, rows_v, sem):
        wid = lax.axis_index("s") * _NC + lax.axis_index("c")
        base = wid * _BPW

        def body(c, carry):
            off = base + c * _CH
            pltpu.sync_copy(idx_hbm.at[pl.ds(off, _CH)], idx_v)
            pltpu.async_copy(tab_hbm.at[idx_v], rows_v, sem).wait()
            pltpu.sync_copy(rows_v, out_hbm.at[pl.ds(off, _CH)])
            return carry

        lax.fori_loop(0, _NCH, body, 0)

    return gk(features, idxp)


def _conv_body(fs_ref, kv_ref, ks_ref, kc_ref, w_ref, g_ref, be_ref, c_ref,
               wm_ref, bm_ref,
               out_ref, a1_ref, a2_ref, m_ref,
               s_ref, carry_ref, sum_ref, sq_ref, sm, sem):
    b = pl.program_id(0)
    nb = pl.num_programs(0)

    @pl.when(b == 0)
    def _init():
        sm[0] = ks_ref[0, 0, 0] >> 3
        sm[1] = 0
        carry_ref[...] = jnp.zeros_like(carry_ref)
        sum_ref[...] = jnp.zeros_like(sum_ref)
        sq_ref[...] = jnp.zeros_like(sq_ref)

    @pl.when(b > 0)
    def _drain():
        pltpu.make_async_copy(s_ref, out_ref.at[pl.ds(0, _SROWS)], sem).wait()

    keys = kv_ref[0] >> 3
    prev = sm[0]
    lane = lax.broadcasted_iota(jnp.int32, (1, _R), 1)
    shifted = jnp.where(lane == 0, prev, pltpu.roll(keys, 1, axis=1))
    f = (keys != shifted).astype(jnp.float32)
    triu = (lax.broadcasted_iota(jnp.int32, (_R, _R), 0)
            <= lax.broadcasted_iota(jnp.int32, (_R, _R), 1)).astype(jnp.float32)
    local = lax.dot(f, triu, preferred_element_type=jnp.float32)
    lf = jnp.max(local)

    fblk = fs_ref[...]
    kc = kc_ref[...] & 7
    p = jnp.zeros((_R, _COUT), jnp.float32)
    for k in range(8):
        mk = (kc == k).astype(jnp.float32)
        p = p + lax.dot(fblk * mk, w_ref[k], preferred_element_type=jnp.float32)

    lsub = lax.broadcasted_iota(jnp.int32, (_SROWS, _R), 0).astype(jnp.float32)
    a = (lsub == local).astype(jnp.float32)
    s = lax.dot(a, p, preferred_element_type=jnp.float32)
    rowi = lax.broadcasted_iota(jnp.int32, (_SROWS, 1), 0).astype(jnp.float32)
    s = s + (rowi == 0.0).astype(jnp.float32) * carry_ref[...]

    is_last = b == nb - 1
    lim = lf + jnp.where(is_last, 1.0, 0.0)
    cm = (rowi < lim).astype(jnp.float32)
    sc = s * cm
    sum_ref[...] += jnp.sum(sc, axis=0, keepdims=True)
    sq_ref[...] += jnp.sum(sc * sc, axis=0, keepdims=True)

    ml = (rowi == lf).astype(jnp.float32)
    carry_ref[...] = jnp.sum(s * ml, axis=0, keepdims=True)

    s_ref[...] = s
    base = sm[1]
    pltpu.make_async_copy(s_ref, out_ref.at[pl.ds(base, _SROWS)], sem).start()

    li = lf.astype(jnp.int32)
    sm[1] = base + li
    sm[0] = ks_ref[0, 0, _R - 1] >> 3

    @pl.when(is_last)
    def _fin():
        pltpu.make_async_copy(s_ref, out_ref.at[pl.ds(0, _SROWS)], sem).wait()
        m_i = base + li + 1
        m_ref[0, 0] = m_i
        mf = m_i.astype(jnp.float32)
        mean = sum_ref[...] / mf
        var = jnp.maximum(sq_ref[...] / mf - mean * mean, 0.0)
        inv = lax.rsqrt(var + 1e-5)
        mod = lax.dot(c_ref[...], wm_ref[...],
                      preferred_element_type=jnp.float32) + bm_ref[...]
        shift = mod[:, :_COUT]
        scale = mod[:, _COUT:]
        a1 = inv * g_ref[...] * (1.0 + scale)
        a1_ref[...] = a1
        a2_ref[...] = (be_ref[...] - mean * inv * g_ref[...]) * (1.0 + scale) \
            + shift


def _conv_tc(fs, keys3, kidx2, w, gamma2, beta2, ctx2, wm, bm2):
    return pl.pallas_call(
        _conv_body,
        grid=(_NB,),
        in_specs=[
            pl.BlockSpec((_R, _CIN), lambda b: (b, 0)),
            pl.BlockSpec((1, 1, _R), lambda b: (b, 0, 0)),
            pl.BlockSpec((1, 1, _R), lambda b: (b, 0, 0),
                         memory_space=pltpu.SMEM),
            pl.BlockSpec((_R, 1), lambda b: (b, 0)),
            pl.BlockSpec((8, _CIN, _COUT), lambda b: (0, 0, 0)),
            pl.BlockSpec((1, _COUT), lambda b: (0, 0)),
            pl.BlockSpec((1, _COUT), lambda b: (0, 0)),
            pl.BlockSpec((1, _CTX), lambda b: (0, 0)),
            pl.BlockSpec((_CTX, 2 * _COUT), lambda b: (0, 0)),
            pl.BlockSpec((1, 2 * _COUT), lambda b: (0, 0)),
        ],
        out_specs=[
            pl.BlockSpec(memory_space=pl.ANY),
            pl.BlockSpec((1, _COUT), lambda b: (0, 0)),
            pl.BlockSpec((1, _COUT), lambda b: (0, 0)),
            pl.BlockSpec(memory_space=pltpu.SMEM),
        ],
        out_shape=[
            jax.ShapeDtypeStruct((_OUT_PAD, _COUT), jnp.float32),
            jax.ShapeDtypeStruct((1, _COUT), jnp.float32),
            jax.ShapeDtypeStruct((1, _COUT), jnp.float32),
            jax.ShapeDtypeStruct((1, 1), jnp.int32),
        ],
        scratch_shapes=[
            pltpu.VMEM((_SROWS, _COUT), jnp.float32),
            pltpu.VMEM((1, _COUT), jnp.float32),
            pltpu.VMEM((1, _COUT), jnp.float32),
            pltpu.VMEM((1, _COUT), jnp.float32),
            pltpu.SMEM((2,), jnp.int32),
            pltpu.SemaphoreType.DMA,
        ],
    )(fs, keys3, keys3, kidx2, w, gamma2, beta2, ctx2, wm, bm2)


_RN = 4000
_NBN = _N // _RN


def _norm_body(o_ref, a1_ref, a2_ref, m_ref, act_ref):
    b = pl.program_id(0)
    m_i = m_ref[0, 0]
    a = o_ref[...] * a1_ref[...] + a2_ref[...]
    rows = lax.broadcasted_iota(jnp.int32, (_RN, 1), 0) + b * _RN
    act_ref[...] = jnp.where(rows < m_i, jnp.maximum(a, 0.0), 0.0)


def _norm_tc(outp, a1, a2, m):
    return pl.pallas_call(
        _norm_body,
        grid=(_NBN,),
        in_specs=[
            pl.BlockSpec((_RN, _COUT), lambda b: (b, 0)),
            pl.BlockSpec((1, _COUT), lambda b: (0, 0)),
            pl.BlockSpec((1, _COUT), lambda b: (0, 0)),
            pl.BlockSpec(memory_space=pltpu.SMEM),
        ],
        out_specs=pl.BlockSpec((_RN, _COUT), lambda b: (b, 0)),
        out_shape=jax.ShapeDtypeStruct((_N, _COUT), jnp.float32),
    )(outp, a1, a2, m)


def kernel(features, coords, W, gamma, beta, context, Wm, bm):
    cb = coords[:, 0]
    cz = coords[:, 1]
    cy = coords[:, 2]
    cx = coords[:, 3]
    key = ((cb * 256 + cz // 2) * 256 + cy // 2) * 32 + cx // 2
    kidx = (cz % 2) * 4 + (cy % 2) * 2 + (cx % 2)
    kb = key * 8 + kidx
    iot = jnp.arange(_N, dtype=jnp.int32)
    kb_s, perm = lax.sort((kb, iot), num_keys=1)
    padidx = jnp.arange(_PADN - _N, dtype=jnp.int32)
    permp = jnp.concatenate([perm, padidx])
    fs = _gather_sorted(features, permp)
    keys3 = kb_s.reshape(_NB, 1, _R)
    kidx2 = kb_s.reshape(_N, 1)
    outp, a1, a2, m = _conv_tc(fs, keys3, kidx2, W,
                               gamma.reshape(1, _COUT), beta.reshape(1, _COUT),
                               context.reshape(1, _CTX), Wm,
                               bm.reshape(1, 2 * _COUT))
    return _norm_tc(outp, a1, a2, m)


# speedup vs baseline: 1.7150x; 1.0845x over previous
import functools

import jax
import jax.numpy as jnp
from jax import lax
from jax.experimental import pallas as pl
from jax.experimental.pallas import tpu as pltpu
from jax.experimental.pallas import tpu_sc as plsc

_N = 200000
_CIN = 32
_COUT = 64
_CTX = 256

_R = 200
_SUB = 4
_STEP = _R * _SUB
_NB = _N // _STEP
_SROWS = 208
_OUT_PAD = _N + _SROWS

_NC = 2
_NS = 16
_NW = _NC * _NS
_CH = 128
_NCH = 49
_BPW = _CH * _NCH
_PADN = _BPW * _NW


def _gather_sorted(features, idxp):
    mesh = plsc.VectorSubcoreMesh(core_axis_name="c", subcore_axis_name="s")

    @functools.partial(
        pl.kernel,
        mesh=mesh,
        compiler_params=pltpu.CompilerParams(use_tc_tiling_on_sc=False),
        out_type=jax.ShapeDtypeStruct((_PADN, _CIN), jnp.float32),
        scratch_types=[
            pltpu.VMEM((_CH,), jnp.int32),
            pltpu.VMEM((_CH, _CIN), jnp.float32),
            pltpu.SemaphoreType.DMA,
        ],
    )
    def gk(tab_hbm, idx_hbm, out_hbm, idx_v, rows_v, sem):
        wid = lax.axis_index("s") * _NC + lax.axis_index("c")
        base = wid * _BPW

        def body(c, carry):
            off = base + c * _CH
            pltpu.sync_copy(idx_hbm.at[pl.ds(off, _CH)], idx_v)
            pltpu.async_copy(tab_hbm.at[idx_v], rows_v, sem).wait()
            pltpu.sync_copy(rows_v, out_hbm.at[pl.ds(off, _CH)])
            return carry

        lax.fori_loop(0, _NCH, body, 0)

    return gk(features, idxp)


def _conv_body(fs_ref, kv_ref, ks_ref, kc_ref, w_ref, g_ref, be_ref, c_ref,
               wm_ref, bm_ref,
               out_ref, a1_ref, a2_ref, m_ref,
               s_refs, carry_ref, sum_ref, sq_ref, sm, sem):
    b = pl.program_id(0)
    nb = pl.num_programs(0)

    @pl.when(b == 0)
    def _init():
        sm[0] = ks_ref[0, 0, 0] >> 3
        sm[1] = 0
        carry_ref[...] = jnp.zeros_like(carry_ref)
        sum_ref[...] = jnp.zeros_like(sum_ref)
        sq_ref[...] = jnp.zeros_like(sq_ref)

    @pl.when(b > 0)
    def _drain():
        for j in range(_SUB):
            pltpu.make_async_copy(s_refs.at[j], out_ref.at[pl.ds(0, _SROWS)],
                                  sem.at[j]).wait()

    krow = kv_ref[0] >> 3
    lane = lax.broadcasted_iota(jnp.int32, (1, _R), 1)
    triu = (lax.broadcasted_iota(jnp.int32, (_R, _R), 0)
            <= lax.broadcasted_iota(jnp.int32, (_R, _R), 1)).astype(jnp.float32)
    lsub = lax.broadcasted_iota(jnp.int32, (_SROWS, _R), 0).astype(jnp.float32)
    rowi = lax.broadcasted_iota(jnp.int32, (_SROWS, 1), 0).astype(jnp.float32)
    row0 = (rowi == 0.0).astype(jnp.float32)

    base = sm[1]
    prev = sm[0]
    carry = carry_ref[...]
    ssum = sum_ref[...]
    ssq = sq_ref[...]

    for j in range(_SUB):
        keys = krow[:, j * _R:(j + 1) * _R]
        shifted = jnp.where(lane == 0, prev, pltpu.roll(keys, 1, axis=1))
        f = (keys != shifted).astype(jnp.float32)
        local = lax.dot(f, triu, preferred_element_type=jnp.float32)
        lf = jnp.max(local)

        fblk = fs_ref[j * _R:(j + 1) * _R, :]
        kc = kc_ref[j * _R:(j + 1) * _R, :] & 7
        p = jnp.zeros((_R, _COUT), jnp.float32)
        for k in range(8):
            mk = (kc == k).astype(jnp.float32)
            p = p + lax.dot(fblk * mk, w_ref[k],
                            preferred_element_type=jnp.float32)

        a = (lsub == local).astype(jnp.float32)
        s = lax.dot(a, p, preferred_element_type=jnp.float32)
        s = s + row0 * carry

        if j == _SUB - 1:
            lim = lf + jnp.where(b == nb - 1, 1.0, 0.0)
        else:
            lim = lf
        cm = (rowi < lim).astype(jnp.float32)
        sc = s * cm
        ssum = ssum + jnp.sum(sc, axis=0, keepdims=True)
        ssq = ssq + jnp.sum(sc * sc, axis=0, keepdims=True)
        carry = jnp.sum(s * (rowi == lf).astype(jnp.float32), axis=0,
                        keepdims=True)

        s_refs[j] = s
        pltpu.make_async_copy(s_refs.at[j], out_ref.at[pl.ds(base, _SROWS)],
                              sem.at[j]).start()
        base = base + lf.astype(jnp.int32)
        prev = ks_ref[0, 0, (j + 1) * _R - 1] >> 3

    sum_ref[...] = ssum
    sq_ref[...] = ssq
    carry_ref[...] = carry
    sm[1] = base
    sm[0] = prev

    @pl.when(b == nb - 1)
    def _fin():
        for j in range(_SUB):
            pltpu.make_async_copy(s_refs.at[j], out_ref.at[pl.ds(0, _SROWS)],
                                  sem.at[j]).wait()
        m_i = base + 1
        m_ref[0, 0] = m_i
        mf = m_i.astype(jnp.float32)
        mean = sum_ref[...] / mf
        var = jnp.maximum(sq_ref[...] / mf - mean * mean, 0.0)
        inv = lax.rsqrt(var + 1e-5)
        mod = lax.dot(c_ref[...], wm_ref[...],
                      preferred_element_type=jnp.float32) + bm_ref[...]
        shift = mod[:, :_COUT]
        scale = mod[:, _COUT:]
        a1_ref[...] = inv * g_ref[...] * (1.0 + scale)
        a2_ref[...] = (be_ref[...] - mean * inv * g_ref[...]) * (1.0 + scale) \
            + shift


def _conv_tc(fs, keys3, kidx2, w, gamma2, beta2, ctx2, wm, bm2):
    return pl.pallas_call(
        _conv_body,
        grid=(_NB,),
        in_specs=[
            pl.BlockSpec((_STEP, _CIN), lambda b: (b, 0)),
            pl.BlockSpec((1, 1, _STEP), lambda b: (b, 0, 0)),
            pl.BlockSpec((1, 1, _STEP), lambda b: (b, 0, 0),
                         memory_space=pltpu.SMEM),
            pl.BlockSpec((_STEP, 1), lambda b: (b, 0)),
            pl.BlockSpec((8, _CIN, _COUT), lambda b: (0, 0, 0)),
            pl.BlockSpec((1, _COUT), lambda b: (0, 0)),
            pl.BlockSpec((1, _COUT), lambda b: (0, 0)),
            pl.BlockSpec((1, _CTX), lambda b: (0, 0)),
            pl.BlockSpec((_CTX, 2 * _COUT), lambda b: (0, 0)),
            pl.BlockSpec((1, 2 * _COUT), lambda b: (0, 0)),
        ],
        out_specs=[
            pl.BlockSpec(memory_space=pl.ANY),
            pl.BlockSpec((1, _COUT), lambda b: (0, 0)),
            pl.BlockSpec((1, _COUT), lambda b: (0, 0)),
            pl.BlockSpec(memory_space=pltpu.SMEM),
        ],
        out_shape=[
            jax.ShapeDtypeStruct((_OUT_PAD, _COUT), jnp.float32),
            jax.ShapeDtypeStruct((1, _COUT), jnp.float32),
            jax.ShapeDtypeStruct((1, _COUT), jnp.float32),
            jax.ShapeDtypeStruct((1, 1), jnp.int32),
        ],
        scratch_shapes=[
            pltpu.VMEM((_SUB, _SROWS, _COUT), jnp.float32),
            pltpu.VMEM((1, _COUT), jnp.float32),
            pltpu.VMEM((1, _COUT), jnp.float32),
            pltpu.VMEM((1, _COUT), jnp.float32),
            pltpu.SMEM((2,), jnp.int32),
            pltpu.SemaphoreType.DMA((_SUB,)),
        ],
    )(fs, keys3, keys3, kidx2, w, gamma2, beta2, ctx2, wm, bm2)


_RN = 4000
_NBN = _N // _RN


def _norm_body(o_ref, a1_ref, a2_ref, m_ref, act_ref):
    b = pl.program_id(0)
    m_i = m_ref[0, 0]
    a = o_ref[...] * a1_ref[...] + a2_ref[...]
    rows = lax.broadcasted_iota(jnp.int32, (_RN, 1), 0) + b * _RN
    act_ref[...] = jnp.where(rows < m_i, jnp.maximum(a, 0.0), 0.0)


def _norm_tc(outp, a1, a2, m):
    return pl.pallas_call(
        _norm_body,
        grid=(_NBN,),
        in_specs=[
            pl.BlockSpec((_RN, _COUT), lambda b: (b, 0)),
            pl.BlockSpec((1, _COUT), lambda b: (0, 0)),
            pl.BlockSpec((1, _COUT), lambda b: (0, 0)),
            pl.BlockSpec(memory_space=pltpu.SMEM),
        ],
        out_specs=pl.BlockSpec((_RN, _COUT), lambda b: (b, 0)),
        out_shape=jax.ShapeDtypeStruct((_N, _COUT), jnp.float32),
    )(outp, a1, a2, m)


def kernel(features, coords, W, gamma, beta, context, Wm, bm):
    cb = coords[:, 0]
    cz = coords[:, 1]
    cy = coords[:, 2]
    cx = coords[:, 3]
    key = ((cb * 256 + cz // 2) * 256 + cy // 2) * 32 + cx // 2
    kidx = (cz % 2) * 4 + (cy % 2) * 2 + (cx % 2)
    kb = key * 8 + kidx
    iot = jnp.arange(_N, dtype=jnp.int32)
    kb_s, perm = lax.sort((kb, iot), num_keys=1)
    padidx = jnp.arange(_PADN - _N, dtype=jnp.int32)
    permp = jnp.concatenate([perm, padidx])
    fs = _gather_sorted(features, permp)
    keys3 = kb_s.reshape(_NB, 1, _STEP)
    kidx2 = kb_s.reshape(_N, 1)
    outp, a1, a2, m = _conv_tc(fs, keys3, kidx2, W,
                               gamma.reshape(1, _COUT), beta.reshape(1, _COUT),
                               context.reshape(1, _CTX), Wm,
                               bm.reshape(1, 2 * _COUT))
    return _norm_tc(outp, a1, a2, m)
